# baseline (speedup 1.0000x reference)
import jax
import jax.numpy as jnp
from jax import lax
from jax.experimental import pallas as pl
from jax.experimental.pallas import tpu as pltpu
from jax.experimental.pallas import tpu_sc as plsc

EMB = 64
B, LSEQ, T = 1024, 50, 20
NC, NS, LANES = 2, 16, 16
NW = NC * NS
NE = EMB // LANES
G = 25
BPW = B // NW
NPAIR = BPW // 2


def _sc_body(seq_hbm, w_hbm, out_hbm,
             idxA, idxB, idxgA, idxgB, rows0, rows1, out0, out1, w0_v,
             gsem0, gsem1, osem0, osem1):
    wid = lax.axis_index("s") * NC + lax.axis_index("c")
    b0 = wid * BPW

    idxs = (idxA, idxB)
    idxgs = (idxgA, idxgB)
    rows = (rows0, rows1)
    outs = (out0, out1)
    gsems = (gsem0, gsem1)
    osems = (osem0, osem1)

    pltpu.sync_copy(w_hbm.at[pl.ds(0, 1)], w0_v)

    def stage_idx(b, p):
        pltpu.sync_copy(seq_hbm.at[b], idxs[p].at[pl.ds(0, LSEQ)])
        for l in range(LSEQ):
            for c0 in (0, T - LANES):
                v = idxs[p][l, pl.ds(c0, LANES)]
                idxgs[p][l, pl.ds(c0, LANES)] = jnp.maximum(v - 1, 0)

    def fire(h, p):
        for l in range(G):
            pltpu.async_copy(
                w_hbm.at[idxgs[p].at[h * G + l]],
                rows[h].at[pl.ds(l * T, T)],
                gsems[h])

    def drain(h, p):
        for l in range(G):
            pltpu.make_async_copy(
                w_hbm.at[idxgs[p].at[h * G + l]],
                rows[h].at[pl.ds(l * T, T)],
                gsems[h]).wait()

    def compute(b, h, p, first):
        out_v = outs[h]
        rows_v = rows[h]
        idx_v = idxs[p]
        osem = osems[h]

        @pl.when(jnp.logical_not(first))
        def _():
            pltpu.make_async_copy(
                out_v, out_hbm.at[b - 1, pl.ds(h * G, G)], osem).wait()

        @plsc.parallel_loop(0, G, 1, unroll=2)
        def group(g):
            w0 = [w0_v[0, pl.ds(e * LANES, LANES)] for e in range(NE)]
            w0m = [lax.bitcast_convert_type(x, jnp.int32) != 0 for x in w0]
            base = g * T
            row = h * G + g
            v1 = idx_v[row, pl.ds(0, LANES)]
            v2 = idx_v[row, pl.ds(T - LANES, LANES)]
            iot = lax.iota(jnp.int32, LANES)
            m1 = v1 == 0
            m2 = jnp.logical_and(v2 == 0, iot >= 2 * LANES - T)
            zcb = jnp.where(m1, 1, 0) + jnp.where(m2, 1, 0)
            dnums = lax.GatherDimensionNumbers(
                offset_dims=(), collapsed_slice_dims=(0,),
                start_index_map=(0,))
            for sh in (8, 4, 2, 1):
                perm = ((iot + sh) & (LANES - 1)).reshape(LANES, 1)
                zcb = zcb + lax.gather(
                    zcb, perm, dnums, (1,),
                    mode=lax.GatherScatterMode.PROMISE_IN_BOUNDS)
            zcf = zcb.astype(jnp.float32)
            s = [jnp.zeros((LANES,), jnp.float32) for _ in range(NE)]
            c = [jnp.zeros((LANES,), jnp.int32) for _ in range(NE)]
            for t in range(T):
                for e in range(NE):
                    r = rows_v[base + t, pl.ds(e * LANES, LANES)]
                    s[e] = s[e] + r
                    bb = lax.bitcast_convert_type(r, jnp.int32)
                    c[e] = jnp.where(bb != 0, c[e] + 1, c[e])
            for e in range(NE):
                se = s[e] - zcf * w0[e]
                ce = c[e] - jnp.where(w0m[e], zcb, 0)
                cf = ce.astype(jnp.float32)
                out_v[g, pl.ds(e * LANES, LANES)] = jnp.where(
                    ce == 0, 0.0, se / cf)

        pltpu.async_copy(out_v, out_hbm.at[b, pl.ds(h * G, G)], osem)

    stage_idx(b0, 0)
    fire(0, 0)
    fire(1, 0)

    def outer(io, carry):
        b = b0 + 2 * io
        for q in range(2):
            bq = b + q
            p = q
            more = (io < NPAIR - 1) if q == 1 else True
            first = (io == 0) if q == 0 else False

            if q == 0:
                stage_idx(bq + 1, 1 - p)
            else:
                @pl.when(io < NPAIR - 1)
                def _():
                    stage_idx(bq + 1, 1 - p)

            for h in range(2):
                drain(h, p)
                compute(bq, h, p, first)
                if q == 0:
                    fire(h, 1 - p)
                else:
                    @pl.when(io < NPAIR - 1)
                    def _(h=h, p=p):
                        fire(h, 1 - p)
        return carry

    lax.fori_loop(0, NPAIR, outer, 0)

    blast = b0 + BPW - 1
    for h in range(2):
        pltpu.make_async_copy(
            outs[h], out_hbm.at[blast, pl.ds(h * G, G)], osems[h]).wait()


def kernel(seq, W):
    mesh = plsc.VectorSubcoreMesh(core_axis_name="c", subcore_axis_name="s")
    out = pl.kernel(
        _sc_body,
        mesh=mesh,
        compiler_params=pltpu.CompilerParams(use_tc_tiling_on_sc=False),
        out_type=jax.ShapeDtypeStruct((B, LSEQ, EMB), jnp.float32),
        scratch_types=[
            pltpu.VMEM((LSEQ, T), jnp.int32),
            pltpu.VMEM((LSEQ, T), jnp.int32),
            pltpu.VMEM((LSEQ, T), jnp.int32),
            pltpu.VMEM((LSEQ, T), jnp.int32),
            pltpu.VMEM((G * T, EMB), jnp.float32),
            pltpu.VMEM((G * T, EMB), jnp.float32),
            pltpu.VMEM((G, EMB), jnp.float32),
            pltpu.VMEM((G, EMB), jnp.float32),
            pltpu.VMEM((1, EMB), jnp.float32),
            pltpu.SemaphoreType.DMA,
            pltpu.SemaphoreType.DMA,
            pltpu.SemaphoreType.DMA,
            pltpu.SemaphoreType.DMA,
        ],
    )(seq, W)
    return out


# speedup vs baseline: 1.0688x; 1.0688x over previous
import jax
import jax.numpy as jnp
from jax import lax
from jax.experimental import pallas as pl
from jax.experimental.pallas import tpu as pltpu
from jax.experimental.pallas import tpu_sc as plsc

EMB = 64
ROWW = 128
B, LSEQ, T = 1024, 50, 20
NC, NS, LANES = 2, 16, 16
NW = NC * NS
NE = EMB // LANES
ROWS = B * LSEQ
G = 16
BLKS = ROWS // G
BPW = BLKS // NW
NPAIR = BPW // 2
IDX_PER_BLK = G * T
IDX_CHUNK = 80
NSUB = IDX_PER_BLK // IDX_CHUNK


def _sc_body(seq_hbm, w_hbm, out_hbm,
             idx0, idx1, rows0, rows1, out0, out1,
             sem0, sem1, osem0, osem1):
    wid = lax.axis_index("s") * NC + lax.axis_index("c")

    bufs = ((idx0, rows0, out0, sem0, osem0),
            (idx1, rows1, out1, sem1, osem1))

    def stage(blk, buf):
        idx_v, rows_v, _, sem, _ = bufs[buf]
        pltpu.sync_copy(seq_hbm.at[blk], idx_v)
        for j in range(NSUB):
            pltpu.async_copy(
                w_hbm.at[idx_v.at[j]],
                rows_v.at[pl.ds(j * IDX_CHUNK, IDX_CHUNK)],
                sem,
            )

    def drain(buf):
        _, rows_v, _, sem, _ = bufs[buf]
        pltpu.make_async_copy(
            w_hbm.at[pl.ds(0, IDX_PER_BLK)], rows_v, sem).wait()

    def compute(blk, buf, first):
        _, rows_v, out_v, _, osem = bufs[buf]

        @pl.when(jnp.logical_not(first))
        def _():
            pltpu.make_async_copy(
                out_v, out_hbm.at[pl.ds((blk - 2) * G, G)], osem).wait()

        @plsc.parallel_loop(0, G, 1, unroll=2)
        def group(g):
            base = g * T
            s = [jnp.zeros((LANES,), jnp.float32) for _ in range(NE)]
            c = [jnp.zeros((LANES,), jnp.int32) for _ in range(NE)]
            for t in range(T):
                for e in range(NE):
                    r = rows_v[base + t, pl.ds(e * LANES, LANES)]
                    s[e] = s[e] + r
                    bb = lax.bitcast_convert_type(r, jnp.int32)
                    c[e] = jnp.where(bb != 0, c[e] + 1, c[e])
            for e in range(NE):
                cf = c[e].astype(jnp.float32)
                out_v[g, pl.ds(e * LANES, LANES)] = jnp.where(
                    c[e] == 0, 0.0, s[e] / cf)

        pltpu.async_copy(out_v, out_hbm.at[pl.ds(blk * G, G)], osem)

    stage(wid * BPW, 0)
    stage(wid * BPW + 1, 1)

    def outer(io, carry):
        blk = wid * BPW + 2 * io
        drain(0)
        compute(blk, 0, first=io == 0)

        @pl.when(io < NPAIR - 1)
        def _():
            stage(blk + 2, 0)

        drain(1)
        compute(blk + 1, 1, first=io == 0)

        @pl.when(io < NPAIR - 1)
        def _():
            stage(blk + 3, 1)

        return carry

    lax.fori_loop(0, NPAIR, outer, 0)
    for buf in range(2):
        _, _, out_v, _, osem = bufs[buf]
        last = wid * BPW + BPW - 2 + buf
        pltpu.make_async_copy(
            out_v, out_hbm.at[pl.ds(last * G, G)], osem).wait()


def kernel(seq, W):
    w_full = jnp.pad(W, ((1, 0), (0, ROWW - EMB)))
    seq3 = seq.reshape(BLKS, NSUB, IDX_CHUNK)
    mesh = plsc.VectorSubcoreMesh(core_axis_name="c", subcore_axis_name="s")
    out = pl.kernel(
        _sc_body,
        mesh=mesh,
        compiler_params=pltpu.CompilerParams(use_tc_tiling_on_sc=False),
        out_type=jax.ShapeDtypeStruct((ROWS, EMB), jnp.float32),
        scratch_types=[
            pltpu.VMEM((NSUB, IDX_CHUNK), jnp.int32),
            pltpu.VMEM((NSUB, IDX_CHUNK), jnp.int32),
            pltpu.VMEM((IDX_PER_BLK, ROWW), jnp.float32),
            pltpu.VMEM((IDX_PER_BLK, ROWW), jnp.float32),
            pltpu.VMEM((G, EMB), jnp.float32),
            pltpu.VMEM((G, EMB), jnp.float32),
            pltpu.SemaphoreType.DMA,
            pltpu.SemaphoreType.DMA,
            pltpu.SemaphoreType.DMA,
            pltpu.SemaphoreType.DMA,
        ],
    )(seq3, w_full)
    return out.reshape(B, LSEQ, EMB)


# speedup vs baseline: 1.2486x; 1.1682x over previous
import jax
import jax.numpy as jnp
from jax import lax
from jax.experimental import pallas as pl
from jax.experimental.pallas import tpu as pltpu
from jax.experimental.pallas import tpu_sc as plsc

EMB = 64
B, LSEQ, T = 1024, 50, 20
NC, NS, LANES = 2, 16, 16
NW = NC * NS
NE = EMB // LANES
ROWS = B * LSEQ
G = 32
BLKS = ROWS // G
BPW = BLKS // NW
NPAIR = BPW // 2
IDX_PER_BLK = G * T
IDX_CHUNK = 128
NSUB = IDX_PER_BLK // IDX_CHUNK


def _sc_body(seq_hbm, w_hbm, out_hbm,
             idx0, idx1, rows0, rows1, out0, out1,
             isem0, isem1, gsem0, gsem1, osem0, osem1):
    wid = lax.axis_index("s") * NC + lax.axis_index("c")

    bufs = ((idx0, rows0, out0, isem0, gsem0, osem0),
            (idx1, rows1, out1, isem1, gsem1, osem1))

    def stage_idx(blk, buf):
        idx_v, _, _, isem, _, _ = bufs[buf]
        pltpu.async_copy(seq_hbm.at[blk], idx_v, isem)

    def fire(blk, buf):
        idx_v, rows_v, _, isem, gsem, _ = bufs[buf]
        pltpu.make_async_copy(seq_hbm.at[blk], idx_v, isem).wait()
        for j in range(NSUB):
            pltpu.async_copy(
                w_hbm.at[idx_v.at[j]],
                rows_v.at[pl.ds(j * IDX_CHUNK, IDX_CHUNK)],
                gsem,
            )

    def drain_rows(buf):
        _, rows_v, _, _, gsem, _ = bufs[buf]
        pltpu.make_async_copy(
            w_hbm.at[pl.ds(0, IDX_PER_BLK)], rows_v, gsem).wait()

    def compute(blk, buf, first):
        _, rows_v, out_v, _, _, osem = bufs[buf]

        @pl.when(jnp.logical_not(first))
        def _():
            pltpu.make_async_copy(
                out_v, out_hbm.at[pl.ds((blk - 2) * G, G)], osem).wait()

        @plsc.parallel_loop(0, G, 1, unroll=2)
        def group(g):
            base = g * T
            s = [jnp.zeros((LANES,), jnp.float32) for _ in range(NE)]
            c = [jnp.zeros((LANES,), jnp.int32) for _ in range(NE)]
            for t in range(T):
                for e in range(NE):
                    r = rows_v[base + t, pl.ds(e * LANES, LANES)]
                    s[e] = s[e] + r
                    bb = lax.bitcast_convert_type(r, jnp.int32)
                    c[e] = jnp.where(bb != 0, c[e] + 1, c[e])
            for e in range(NE):
                cf = c[e].astype(jnp.float32)
                out_v[g, pl.ds(e * LANES, LANES)] = jnp.where(
                    c[e] == 0, 0.0, s[e] / cf)

        pltpu.async_copy(out_v, out_hbm.at[pl.ds(blk * G, G)], osem)

    base_blk = wid * BPW
    stage_idx(base_blk, 0)
    stage_idx(base_blk + 1, 1)
    fire(base_blk, 0)

    def outer(io, carry):
        blk = base_blk + 2 * io

        drain_rows(0)

        @pl.when(io < NPAIR - 1)
        def _():
            stage_idx(blk + 2, 0)

        fire(blk + 1, 1)
        compute(blk, 0, first=io == 0)

        drain_rows(1)

        @pl.when(io < NPAIR - 1)
        def _():
            stage_idx(blk + 3, 1)
            fire(blk + 2, 0)

        compute(blk + 1, 1, first=io == 0)
        return carry

    lax.fori_loop(0, NPAIR, outer, 0)
    for buf in range(2):
        _, _, out_v, _, _, osem = bufs[buf]
        last = base_blk + BPW - 2 + buf
        pltpu.make_async_copy(
            out_v, out_hbm.at[pl.ds(last * G, G)], osem).wait()


def kernel(seq, W):
    w_full = jnp.concatenate([jnp.zeros((1, EMB), W.dtype), W], axis=0)
    seq3 = seq.reshape(BLKS, NSUB, IDX_CHUNK)
    mesh = plsc.VectorSubcoreMesh(core_axis_name="c", subcore_axis_name="s")
    out = pl.kernel(
        _sc_body,
        mesh=mesh,
        compiler_params=pltpu.CompilerParams(use_tc_tiling_on_sc=False),
        out_type=jax.ShapeDtypeStruct((ROWS, EMB), jnp.float32),
        scratch_types=[
            pltpu.VMEM((NSUB, IDX_CHUNK), jnp.int32),
            pltpu.VMEM((NSUB, IDX_CHUNK), jnp.int32),
            pltpu.VMEM((IDX_PER_BLK, EMB), jnp.float32),
            pltpu.VMEM((IDX_PER_BLK, EMB), jnp.float32),
            pltpu.VMEM((G, EMB), jnp.float32),
            pltpu.VMEM((G, EMB), jnp.float32),
            pltpu.SemaphoreType.DMA,
            pltpu.SemaphoreType.DMA,
            pltpu.SemaphoreType.DMA,
            pltpu.SemaphoreType.DMA,
            pltpu.SemaphoreType.DMA,
            pltpu.SemaphoreType.DMA,
        ],
    )(seq3, w_full)
    return out.reshape(B, LSEQ, EMB)


# speedup vs baseline: 1.2663x; 1.0141x over previous
import jax
import jax.numpy as jnp
from jax import lax
from jax.experimental import pallas as pl
from jax.experimental.pallas import tpu as pltpu
from jax.experimental.pallas import tpu_sc as plsc

EMB = 64
B, LSEQ, T = 1024, 50, 20
NC, NS, LANES = 2, 16, 16
NW = NC * NS
NE = EMB // LANES
ROWS = B * LSEQ
G = 40
BLKS = ROWS // G
BPW = BLKS // NW
NPAIR = BPW // 2
IDX_PER_BLK = G * T
IDX_CHUNK = 80
NSUB = IDX_PER_BLK // IDX_CHUNK


def _sc_body(seq_hbm, w_hbm, out_hbm,
             idx0, idx1, rows0, rows1, out0, out1,
             isem0, isem1, gsem0, gsem1, osem0, osem1):
    wid = lax.axis_index("s") * NC + lax.axis_index("c")

    bufs = ((idx0, rows0, out0, isem0, gsem0, osem0),
            (idx1, rows1, out1, isem1, gsem1, osem1))

    def stage_idx(blk, buf):
        idx_v, _, _, isem, _, _ = bufs[buf]
        pltpu.async_copy(seq_hbm.at[blk], idx_v, isem)

    def fire(blk, buf):
        idx_v, rows_v, _, isem, gsem, _ = bufs[buf]
        pltpu.make_async_copy(seq_hbm.at[blk], idx_v, isem).wait()
        for j in range(NSUB):
            pltpu.async_copy(
                w_hbm.at[idx_v.at[j]],
                rows_v.at[pl.ds(j * IDX_CHUNK, IDX_CHUNK)],
                gsem,
            )

    def drain_rows(buf):
        _, rows_v, _, _, gsem, _ = bufs[buf]
        pltpu.make_async_copy(
            w_hbm.at[pl.ds(0, IDX_PER_BLK)], rows_v, gsem).wait()

    def compute(blk, buf, first):
        _, rows_v, out_v, _, _, osem = bufs[buf]

        @pl.when(jnp.logical_not(first))
        def _():
            pltpu.make_async_copy(
                out_v, out_hbm.at[pl.ds((blk - 2) * G, G)], osem).wait()

        @plsc.parallel_loop(0, G, 1, unroll=2)
        def group(g):
            base = g * T
            s = [jnp.zeros((LANES,), jnp.float32) for _ in range(NE)]
            c = [jnp.zeros((LANES,), jnp.int32) for _ in range(NE)]
            for t in range(T):
                for e in range(NE):
                    r = rows_v[base + t, pl.ds(e * LANES, LANES)]
                    s[e] = s[e] + r
                    bb = lax.bitcast_convert_type(r, jnp.int32)
                    c[e] = c[e] + jnp.where(bb != 0, 1, 0)
            for e in range(NE):
                cf = c[e].astype(jnp.float32)
                out_v[g, pl.ds(e * LANES, LANES)] = jnp.where(
                    c[e] == 0, 0.0, s[e] / cf)

        pltpu.async_copy(out_v, out_hbm.at[pl.ds(blk * G, G)], osem)

    base_blk = wid * BPW
    stage_idx(base_blk, 0)
    stage_idx(base_blk + 1, 1)
    fire(base_blk, 0)

    def outer(io, carry):
        blk = base_blk + 2 * io

        drain_rows(0)

        @pl.when(io < NPAIR - 1)
        def _():
            stage_idx(blk + 2, 0)

        fire(blk + 1, 1)
        compute(blk, 0, first=io == 0)

        drain_rows(1)

        @pl.when(io < NPAIR - 1)
        def _():
            stage_idx(blk + 3, 1)
            fire(blk + 2, 0)

        compute(blk + 1, 1, first=io == 0)
        return carry

    lax.fori_loop(0, NPAIR, outer, 0)
    for buf in range(2):
        _, _, out_v, _, _, osem = bufs[buf]
        last = base_blk + BPW - 2 + buf
        pltpu.make_async_copy(
            out_v, out_hbm.at[pl.ds(last * G, G)], osem).wait()


def kernel(seq, W):
    w_full = jnp.concatenate([jnp.zeros((1, EMB), W.dtype), W], axis=0)
    seq3 = seq.reshape(BLKS, NSUB, IDX_CHUNK)
    mesh = plsc.VectorSubcoreMesh(core_axis_name="c", subcore_axis_name="s")
    out = pl.kernel(
        _sc_body,
        mesh=mesh,
        compiler_params=pltpu.CompilerParams(use_tc_tiling_on_sc=False),
        out_type=jax.ShapeDtypeStruct((ROWS, EMB), jnp.float32),
        scratch_types=[
            pltpu.VMEM((NSUB, IDX_CHUNK), jnp.int32),
            pltpu.VMEM((NSUB, IDX_CHUNK), jnp.int32),
            pltpu.VMEM((IDX_PER_BLK, EMB), jnp.float32),
            pltpu.VMEM((IDX_PER_BLK, EMB), jnp.float32),
            pltpu.VMEM((G, EMB), jnp.float32),
            pltpu.VMEM((G, EMB), jnp.float32),
            pltpu.SemaphoreType.DMA,
            pltpu.SemaphoreType.DMA,
            pltpu.SemaphoreType.DMA,
            pltpu.SemaphoreType.DMA,
            pltpu.SemaphoreType.DMA,
            pltpu.SemaphoreType.DMA,
        ],
    )(seq3, w_full)
    return out.reshape(B, LSEQ, EMB)
